# Initial kernel scaffold; baseline (speedup 1.0000x reference)
import jax
import jax.numpy as jnp
from jax.experimental import pallas as pl


def kernel(x, edge_index, W1, b1, W2, b2):
    raise NotImplementedError("write your pallas kernel here")


# speedup vs baseline: 14.5172x; 14.5172x over previous
import functools

import jax
import jax.numpy as jnp
from jax import lax
from jax.experimental import pallas as pl
from jax.experimental.pallas import tpu as pltpu
from jax.experimental.pallas import tpu_sc as plsc

_NC = 2
_NS = 16
_NW = _NC * _NS
_L = 16
_CH = 128
_DEGW = 16


def _sc_mesh():
    return plsc.VectorSubcoreMesh(core_axis_name="c", subcore_axis_name="s")


def _rows_per_sub(n):
    return 8 * (-(-(-(-n // 8)) // _NS))


def _sub_base(sid, n, rp):
    return pl.multiple_of(jnp.minimum(sid * rp, n - rp), 8)


def _zero_acc(zeros_v, acc, base_n, rp):
    full, tail = divmod(rp, _CH)
    for j in range(full):
        pltpu.sync_copy(zeros_v.at[pl.ds(0, _CH)],
                        acc.at[pl.ds(pl.multiple_of(base_n + j * _CH, 8), _CH)])
    if tail:
        pltpu.sync_copy(zeros_v.at[pl.ds(0, tail)],
                        acc.at[pl.ds(pl.multiple_of(base_n + full * _CH, 8), tail)])


def _writeback(acc, out_hbm, cid, base_n, rp):
    full, tail = divmod(rp, _CH)
    for j in range(full):
        r0 = pl.multiple_of(base_n + j * _CH, 8)
        pltpu.sync_copy(acc.at[pl.ds(r0, _CH)], out_hbm.at[cid, pl.ds(r0, _CH)])
    if tail:
        r0 = pl.multiple_of(base_n + full * _CH, 8)
        pltpu.sync_copy(acc.at[pl.ds(r0, tail)], out_hbm.at[cid, pl.ds(r0, tail)])


def _make_deg(n, e):
    ew = e // _NW
    n_full, tail = divmod(ew, _CH)
    rp = _rows_per_sub(n)

    @functools.partial(
        pl.kernel,
        mesh=_sc_mesh(),
        compiler_params=pltpu.CompilerParams(use_tc_tiling_on_sc=False),
        out_type=jax.ShapeDtypeStruct((_NC, n, _DEGW), jnp.float32),
        scratch_types=[
            pltpu.VMEM((_CH,), jnp.int32),
            pltpu.VMEM((max(tail, 1),), jnp.int32),
            pltpu.VMEM((_CH, _DEGW), jnp.float32),
            pltpu.VMEM((_CH, _DEGW), jnp.float32),
            pltpu.VMEM_SHARED((n, _DEGW), jnp.float32),
        ],
    )
    def deg_kernel(col_hbm, out_hbm, idxc, idxc_t, ones_v, zeros_v, acc):
        cid = lax.axis_index("c")
        sid = lax.axis_index("s")
        wid = sid * _NC + cid

        ones = jnp.ones((_L,), jnp.float32)
        zero = jnp.zeros((_L,), jnp.float32)

        def fill(i, _):
            ones_v[i, pl.ds(0, _L)] = ones
            zeros_v[i, pl.ds(0, _L)] = zero
            return 0

        lax.fori_loop(0, _CH, fill, 0)

        base_n = _sub_base(sid, n, rp)
        _zero_acc(zeros_v, acc, base_n, rp)
        plsc.subcore_barrier()

        ebase = wid * ew

        def step(k, _):
            off = pl.multiple_of(ebase + k * _CH, 8)
            pltpu.sync_copy(col_hbm.at[pl.ds(off, _CH)], idxc)
            pltpu.sync_copy(ones_v, acc.at[idxc], add=True)
            return 0

        lax.fori_loop(0, n_full, step, 0)
        if tail:
            off = pl.multiple_of(ebase + n_full * _CH, 8)
            pltpu.sync_copy(col_hbm.at[pl.ds(off, tail)], idxc_t)
            pltpu.sync_copy(ones_v.at[pl.ds(0, tail)], acc.at[idxc_t], add=True)
        plsc.subcore_barrier()

        _writeback(acc, out_hbm, cid, base_n, rp)

    return deg_kernel


def _make_agg(n, d, e):
    ew = e // _NW
    n_full, tail = divmod(ew, _CH)
    rp = _rows_per_sub(n)

    @functools.partial(
        pl.kernel,
        mesh=_sc_mesh(),
        compiler_params=pltpu.CompilerParams(use_tc_tiling_on_sc=False),
        out_type=jax.ShapeDtypeStruct((_NC, n, d), jnp.float32),
        scratch_types=[
            pltpu.VMEM((_CH,), jnp.int32),
            pltpu.VMEM((_CH,), jnp.int32),
            pltpu.VMEM((max(tail, 1),), jnp.int32),
            pltpu.VMEM((max(tail, 1),), jnp.int32),
            pltpu.VMEM((_CH, d), jnp.float32),
            pltpu.VMEM_SHARED((n, d), jnp.float32),
            pltpu.SemaphoreType.DMA,
        ],
    )
    def agg_kernel(g_hbm, row_hbm, col_hbm, out_hbm,
                   idxr, idxc, idxr_t, idxc_t, rows_v, acc, sem):
        cid = lax.axis_index("c")
        sid = lax.axis_index("s")
        wid = sid * _NC + cid

        zero = jnp.zeros((_L,), jnp.float32)

        def zrow(i, _):
            def zcol(j, _):
                rows_v[i, pl.ds(j * _L, _L)] = zero
                return 0
            return lax.fori_loop(0, d // _L, zcol, 0)

        lax.fori_loop(0, _CH, zrow, 0)

        base_n = _sub_base(sid, n, rp)
        _zero_acc(rows_v, acc, base_n, rp)
        plsc.subcore_barrier()

        ebase = wid * ew

        def step(k, _):
            off = pl.multiple_of(ebase + k * _CH, 8)
            pltpu.sync_copy(row_hbm.at[pl.ds(off, _CH)], idxr)
            pltpu.sync_copy(col_hbm.at[pl.ds(off, _CH)], idxc)
            pltpu.async_copy(g_hbm.at[idxr], rows_v, sem).wait()
            pltpu.sync_copy(rows_v, acc.at[idxc], add=True)
            return 0

        lax.fori_loop(0, n_full, step, 0)
        if tail:
            off = pl.multiple_of(ebase + n_full * _CH, 8)
            pltpu.sync_copy(row_hbm.at[pl.ds(off, tail)], idxr_t)
            pltpu.sync_copy(col_hbm.at[pl.ds(off, tail)], idxc_t)
            pltpu.async_copy(g_hbm.at[idxr_t], rows_v.at[pl.ds(0, tail)], sem).wait()
            pltpu.sync_copy(rows_v.at[pl.ds(0, tail)], acc.at[idxc_t], add=True)
        plsc.subcore_barrier()

        _writeback(acc, out_hbm, cid, base_n, rp)

    return agg_kernel


_BN = 1000


def _mm_scale(x, w, degp):
    n, d_in = x.shape
    d_h = w.shape[1]

    def body(x_ref, w_ref, dp_ref, g_ref, dinv_ref):
        deg = dp_ref[0, :, 0:1] + dp_ref[1, :, 0:1]
        pos = deg > 0.0
        dinv = jnp.where(pos, lax.rsqrt(jnp.where(pos, deg, 1.0)), 0.0)
        h = jnp.dot(x_ref[...], w_ref[...], preferred_element_type=jnp.float32)
        g_ref[...] = h * dinv
        dinv_ref[...] = dinv

    return pl.pallas_call(
        body,
        grid=(n // _BN,),
        in_specs=[
            pl.BlockSpec((_BN, d_in), lambda i: (i, 0)),
            pl.BlockSpec((d_in, d_h), lambda i: (0, 0)),
            pl.BlockSpec((_NC, _BN, _DEGW), lambda i: (0, i, 0)),
        ],
        out_specs=[
            pl.BlockSpec((_BN, d_h), lambda i: (i, 0)),
            pl.BlockSpec((_BN, 1), lambda i: (i, 0)),
        ],
        out_shape=[
            jax.ShapeDtypeStruct((n, d_h), jnp.float32),
            jax.ShapeDtypeStruct((n, 1), jnp.float32),
        ],
    )(x, w, degp)


def _mid_layer(parts, dinv, b, w):
    d_h = parts.shape[2]
    n = parts.shape[1]
    d_out = w.shape[1]

    def body(p_ref, dinv_ref, b_ref, w_ref, o_ref):
        h = (p_ref[0] + p_ref[1]) * dinv_ref[...] + b_ref[...]
        h = jnp.maximum(h, 0.0)
        o_ref[...] = jnp.dot(h, w_ref[...],
                             preferred_element_type=jnp.float32) * dinv_ref[...]

    return pl.pallas_call(
        body,
        grid=(n // _BN,),
        in_specs=[
            pl.BlockSpec((_NC, _BN, d_h), lambda i: (0, i, 0)),
            pl.BlockSpec((_BN, 1), lambda i: (i, 0)),
            pl.BlockSpec((1, d_h), lambda i: (0, 0)),
            pl.BlockSpec((d_h, d_out), lambda i: (0, 0)),
        ],
        out_specs=pl.BlockSpec((_BN, d_out), lambda i: (i, 0)),
        out_shape=jax.ShapeDtypeStruct((n, d_out), jnp.float32),
    )(parts, dinv, b, w)


def _final_layer(parts, dinv, b):
    n = parts.shape[1]
    d_out = parts.shape[2]

    def body(p_ref, dinv_ref, b_ref, o_ref):
        h = (p_ref[0] + p_ref[1]) * dinv_ref[...] + b_ref[...]
        m = jnp.max(h, axis=1, keepdims=True)
        ex = jnp.exp(h - m)
        s = jnp.sum(ex, axis=1, keepdims=True)
        o_ref[...] = h - m - jnp.log(s)

    return pl.pallas_call(
        body,
        grid=(n // _BN,),
        in_specs=[
            pl.BlockSpec((_NC, _BN, d_out), lambda i: (0, i, 0)),
            pl.BlockSpec((_BN, 1), lambda i: (i, 0)),
            pl.BlockSpec((1, d_out), lambda i: (0, 0)),
        ],
        out_specs=pl.BlockSpec((_BN, d_out), lambda i: (i, 0)),
        out_shape=jax.ShapeDtypeStruct((n, d_out), jnp.float32),
    )(parts, dinv, b)


def kernel(x, edge_index, W1, b1, W2, b2):
    n, _ = x.shape
    e = edge_index.shape[1]
    row = edge_index[0]
    col = edge_index[1]

    degp = _make_deg(n, e)(col)
    g1, dinv = _mm_scale(x, W1, degp)
    p1 = _make_agg(n, W1.shape[1], e)(g1, row, col)
    g2 = _mid_layer(p1, dinv, b1.reshape(1, -1), W2)
    p2 = _make_agg(n, W2.shape[1], e)(g2, row, col)
    return _final_layer(p2, dinv, b2.reshape(1, -1))
